# baseline (speedup 1.0000x reference)
import functools

import jax
import jax.numpy as jnp
from jax import lax
from jax.experimental import pallas as pl
from jax.experimental.pallas import tpu as pltpu
from jax.experimental.pallas import tpu_sc as plsc

_B, _K, _N, _D, _A = 16384, 16, 50000, 256, 64
_NW = 32
_CH = 128


def _sc_gather(table, idx3, nc):
    H = _D // 2
    mesh = plsc.VectorSubcoreMesh(core_axis_name="c", subcore_axis_name="s")

    @functools.partial(
        pl.kernel,
        out_type=jax.ShapeDtypeStruct((_NW * nc * _CH, H), jnp.int32),
        mesh=mesh,
        scratch_types=[
            pltpu.VMEM((nc, _CH), jnp.int32),
            pltpu.VMEM((_CH, H), jnp.int32),
            pltpu.VMEM((_CH, H), jnp.int32),
            pltpu.SemaphoreType.DMA,
            pltpu.SemaphoreType.DMA,
        ],
    )
    def k(table_hbm, idx_hbm, out_hbm, idx_v, rows0, rows1, sem0, sem1):
        wid = lax.axis_index("s") * 2 + lax.axis_index("c")
        base = wid * nc * _CH
        pltpu.sync_copy(idx_hbm.at[wid], idx_v)
        bufs = (rows0, rows1)
        sems = (sem0, sem1)
        pltpu.async_copy(table_hbm.at[idx_v.at[0]], rows0, sem0)

        @pl.loop(0, nc)
        def _(c):
            slot = lax.rem(c, 2)

            @pl.when(c + 1 < nc)
            def _():
                nxt = lax.rem(c + 1, 2)
                for j in range(2):
                    @pl.when(nxt == j)
                    def _():
                        pltpu.async_copy(
                            table_hbm.at[idx_v.at[c + 1]], bufs[j], sems[j])

            for j in range(2):
                @pl.when(slot == j)
                def _():
                    pltpu.make_async_copy(
                        table_hbm.at[idx_v.at[c]], bufs[j], sems[j]).wait()
                    pltpu.sync_copy(
                        bufs[j], out_hbm.at[pl.ds(base + c * _CH, _CH)])

    return k(table, idx3)


def _tc_attention(rows3, center, conf, Wq, Wk, Wg1, Wg2, bg2, gamma2, beta2):
    RB = 1024
    nb = rows3.shape[0] // _K
    grid = (nb // RB,)

    H = _D // 2

    def body(rows_ref, center_ref, conf_ref, wq_ref, wk_ref, wg1_ref,
             wg2_ref, bg_ref, g_ref, b_ref, out_ref):
        iv = rows_ref[...]
        lo = lax.bitcast_convert_type(jnp.left_shift(iv, 16), jnp.float32)
        hi = lax.bitcast_convert_type(iv, jnp.float32)
        center = center_ref[...]
        q = jnp.dot(center, wq_ref[...], preferred_element_type=jnp.float32)
        wk = wk_ref[...]
        k2 = (jnp.dot(lo, wk[:H], preferred_element_type=jnp.float32)
              + jnp.dot(hi, wk[H:], preferred_element_type=jnp.float32))
        k3 = k2.reshape(RB, _K, _A)
        scores = jnp.sum(k3 * q[:, None, :], axis=-1) * (1.0 / _K ** 0.5)
        st = scores.T
        m = jnp.max(st, axis=0, keepdims=True)
        e = jnp.maximum(conf_ref[...], 4.5399929762484854e-05) * jnp.exp(st - m)
        wt = e / jnp.sum(e, axis=0, keepdims=True)
        w = wt.T
        w3 = w[:, :, None]
        ctx_lo = jnp.sum(w3 * lo.reshape(RB, _K, H), axis=1)
        ctx_hi = jnp.sum(w3 * hi.reshape(RB, _K, H), axis=1)
        wg2 = wg2_ref[...]
        gs = (jnp.dot(center, wg1_ref[...], preferred_element_type=jnp.float32)
              + jnp.dot(ctx_lo, wg2[:H], preferred_element_type=jnp.float32)
              + jnp.dot(ctx_hi, wg2[H:], preferred_element_type=jnp.float32)
              + bg_ref[0, 0])
        gate = 1.0 / (1.0 + jnp.exp(-gs))
        o_lo = gate * center[:, :H] + (1.0 - gate) * ctx_lo
        o_hi = gate * center[:, H:] + (1.0 - gate) * ctx_hi
        mean = (jnp.sum(o_lo, -1, keepdims=True)
                + jnp.sum(o_hi, -1, keepdims=True)) * (1.0 / _D)
        c_lo = o_lo - mean
        c_hi = o_hi - mean
        var = (jnp.sum(c_lo * c_lo, -1, keepdims=True)
               + jnp.sum(c_hi * c_hi, -1, keepdims=True)) * (1.0 / _D)
        rs = lax.rsqrt(var + 1e-5)
        g_all = g_ref[...]
        b_all = b_ref[...]
        out_ref[:, :H] = c_lo * rs * g_all[:, :H] + b_all[:, :H]
        out_ref[:, H:] = c_hi * rs * g_all[:, H:] + b_all[:, H:]

    return pl.pallas_call(
        body,
        grid=grid,
        in_specs=[
            pl.BlockSpec((RB * _K, _D // 2), lambda i: (i, 0)),
            pl.BlockSpec((RB, _D), lambda i: (i, 0)),
            pl.BlockSpec((_K, RB), lambda i: (0, i)),
            pl.BlockSpec((_D, _A), lambda i: (0, 0)),
            pl.BlockSpec((_D, _A), lambda i: (0, 0)),
            pl.BlockSpec((_D, 1), lambda i: (0, 0)),
            pl.BlockSpec((_D, 1), lambda i: (0, 0)),
            pl.BlockSpec((1, 1), lambda i: (0, 0)),
            pl.BlockSpec((1, _D), lambda i: (0, 0)),
            pl.BlockSpec((1, _D), lambda i: (0, 0)),
        ],
        out_specs=pl.BlockSpec((RB, _D), lambda i: (i, 0)),
        out_shape=jax.ShapeDtypeStruct((nb, _D), jnp.float32),
        compiler_params=pltpu.CompilerParams(
            dimension_semantics=("arbitrary",),
        ),
    )(rows3, center, conf, Wq, Wk, Wg1, Wg2, bg2, gamma2, beta2)


def kernel(center_emb, node_embs, neighbor_idx, neighbor_conf, Wq, Wk, Wg,
           bg, gamma, beta):
    G = 4
    BG = _B // G
    nc = (BG * _K) // (_NW * _CH)
    Wg1 = Wg[:_D]
    Wg2 = Wg[_D:]
    bg2 = bg.reshape(1, 1)
    gamma2 = gamma.reshape(1, _D)
    beta2 = beta.reshape(1, _D)
    conf_t = neighbor_conf.T
    nb = node_embs.astype(jnp.bfloat16)
    node_p = lax.bitcast_convert_type(
        jnp.stack([nb[:, :_D // 2], nb[:, _D // 2:]], axis=-1), jnp.int32)
    rows_g = []
    for g in range(G):
        idx3 = neighbor_idx[g * BG:(g + 1) * BG].reshape(_NW, nc, _CH)
        rows_g.append(_sc_gather(node_p, idx3, nc))
    outs = []
    for g in range(G):
        outs.append(_tc_attention(
            rows_g[g], center_emb[g * BG:(g + 1) * BG],
            conf_t[:, g * BG:(g + 1) * BG], Wq, Wk, Wg1, Wg2,
            bg2, gamma2, beta2))
    return jnp.concatenate(outs, axis=0)


# speedup vs baseline: 1.1275x; 1.1275x over previous
import functools

import jax
import jax.numpy as jnp
from jax import lax
from jax.experimental import pallas as pl
from jax.experimental.pallas import tpu as pltpu
from jax.experimental.pallas import tpu_sc as plsc

_B, _K, _N, _D, _A = 16384, 16, 50000, 256, 64
_NW = 32
_CH = 128


def _sc_gather(table, idx3, nc):
    H = _D // 2
    mesh = plsc.VectorSubcoreMesh(core_axis_name="c", subcore_axis_name="s")

    @functools.partial(
        pl.kernel,
        out_type=jax.ShapeDtypeStruct((_NW * nc * _CH, H), jnp.int32),
        mesh=mesh,
        scratch_types=[
            pltpu.VMEM((nc, _CH), jnp.int32),
            pltpu.VMEM((_CH, H), jnp.int32),
            pltpu.VMEM((_CH, H), jnp.int32),
            pltpu.SemaphoreType.DMA,
            pltpu.SemaphoreType.DMA,
        ],
    )
    def k(table_hbm, idx_hbm, out_hbm, idx_v, rows0, rows1, sem0, sem1):
        wid = lax.axis_index("s") * 2 + lax.axis_index("c")
        base = wid * nc * _CH
        pltpu.sync_copy(idx_hbm.at[wid], idx_v)
        bufs = (rows0, rows1)
        sems = (sem0, sem1)
        pltpu.async_copy(table_hbm.at[idx_v.at[0]], rows0, sem0)

        @pl.loop(0, nc)
        def _(c):
            slot = lax.rem(c, 2)

            @pl.when(c + 1 < nc)
            def _():
                nxt = lax.rem(c + 1, 2)
                for j in range(2):
                    @pl.when(nxt == j)
                    def _():
                        pltpu.async_copy(
                            table_hbm.at[idx_v.at[c + 1]], bufs[j], sems[j])

            for j in range(2):
                @pl.when(slot == j)
                def _():
                    pltpu.make_async_copy(
                        table_hbm.at[idx_v.at[c]], bufs[j], sems[j]).wait()
                    pltpu.sync_copy(
                        bufs[j], out_hbm.at[pl.ds(base + c * _CH, _CH)])

    return k(table, idx3)


def _tc_attention(rows3, center, conf, Wq, Wk, Wg1, Wg2, bg2, gamma2, beta2):
    RB = 512
    nb = rows3.shape[0] // _K
    grid = (nb // RB,)

    H = _D // 2

    def body(rows_ref, center_ref, conf_ref, wq_ref, wk_ref, wg1_ref,
             wg2_ref, bg_ref, g_ref, b_ref, out_ref):
        iv = rows_ref[...]
        lo = lax.bitcast_convert_type(jnp.left_shift(iv, 16), jnp.float32)
        hi = lax.bitcast_convert_type(iv, jnp.float32)
        center = center_ref[...]
        q = jnp.dot(center, wq_ref[...], preferred_element_type=jnp.float32)
        wk = wk_ref[...]
        k2 = (jnp.dot(lo, wk[:H], preferred_element_type=jnp.float32)
              + jnp.dot(hi, wk[H:], preferred_element_type=jnp.float32))
        k3 = k2.reshape(RB, _K, _A)
        scores = jnp.sum(k3 * q[:, None, :], axis=-1) * (1.0 / _K ** 0.5)
        st = scores.T
        m = jnp.max(st, axis=0, keepdims=True)
        e = jnp.maximum(conf_ref[...], 4.5399929762484854e-05) * jnp.exp(st - m)
        wt = e / jnp.sum(e, axis=0, keepdims=True)
        w = wt.T
        w3 = w[:, :, None]
        ctx_lo = jnp.sum(w3 * lo.reshape(RB, _K, H), axis=1)
        ctx_hi = jnp.sum(w3 * hi.reshape(RB, _K, H), axis=1)
        wg2 = wg2_ref[...]
        gs = (jnp.dot(center, wg1_ref[...], preferred_element_type=jnp.float32)
              + jnp.dot(ctx_lo, wg2[:H], preferred_element_type=jnp.float32)
              + jnp.dot(ctx_hi, wg2[H:], preferred_element_type=jnp.float32)
              + bg_ref[0, 0])
        gate = 1.0 / (1.0 + jnp.exp(-gs))
        o_lo = gate * center[:, :H] + (1.0 - gate) * ctx_lo
        o_hi = gate * center[:, H:] + (1.0 - gate) * ctx_hi
        mean = (jnp.sum(o_lo, -1, keepdims=True)
                + jnp.sum(o_hi, -1, keepdims=True)) * (1.0 / _D)
        c_lo = o_lo - mean
        c_hi = o_hi - mean
        var = (jnp.sum(c_lo * c_lo, -1, keepdims=True)
               + jnp.sum(c_hi * c_hi, -1, keepdims=True)) * (1.0 / _D)
        rs = lax.rsqrt(var + 1e-5)
        g_all = g_ref[...]
        b_all = b_ref[...]
        out_ref[:, :H] = c_lo * rs * g_all[:, :H] + b_all[:, :H]
        out_ref[:, H:] = c_hi * rs * g_all[:, H:] + b_all[:, H:]

    return pl.pallas_call(
        body,
        grid=grid,
        in_specs=[
            pl.BlockSpec((RB * _K, _D // 2), lambda i: (i, 0)),
            pl.BlockSpec((RB, _D), lambda i: (i, 0)),
            pl.BlockSpec((_K, RB), lambda i: (0, i)),
            pl.BlockSpec((_D, _A), lambda i: (0, 0)),
            pl.BlockSpec((_D, _A), lambda i: (0, 0)),
            pl.BlockSpec((_D, 1), lambda i: (0, 0)),
            pl.BlockSpec((_D, 1), lambda i: (0, 0)),
            pl.BlockSpec((1, 1), lambda i: (0, 0)),
            pl.BlockSpec((1, _D), lambda i: (0, 0)),
            pl.BlockSpec((1, _D), lambda i: (0, 0)),
        ],
        out_specs=pl.BlockSpec((RB, _D), lambda i: (i, 0)),
        out_shape=jax.ShapeDtypeStruct((nb, _D), jnp.float32),
        compiler_params=pltpu.CompilerParams(
            dimension_semantics=("arbitrary",),
        ),
    )(rows3, center, conf, Wq, Wk, Wg1, Wg2, bg2, gamma2, beta2)


def kernel(center_emb, node_embs, neighbor_idx, neighbor_conf, Wq, Wk, Wg,
           bg, gamma, beta):
    G = 4
    BG = _B // G
    nc = (BG * _K) // (_NW * _CH)
    Wg1 = Wg[:_D]
    Wg2 = Wg[_D:]
    bg2 = bg.reshape(1, 1)
    gamma2 = gamma.reshape(1, _D)
    beta2 = beta.reshape(1, _D)
    conf_t = neighbor_conf.T
    iN = lax.bitcast_convert_type(node_embs, jnp.int32)
    rN = iN + 0x7FFF + jnp.bitwise_and(jnp.right_shift(iN, 16), 1)
    node_p = jnp.bitwise_or(
        jnp.bitwise_and(rN[:, _D // 2:], jnp.int32(-65536)),
        jnp.right_shift(rN[:, :_D // 2], 16) & 0xFFFF)
    rows_g = []
    for g in range(G):
        idx3 = neighbor_idx[g * BG:(g + 1) * BG].reshape(_NW, nc, _CH)
        rows_g.append(_sc_gather(node_p, idx3, nc))
    outs = []
    for g in range(G):
        outs.append(_tc_attention(
            rows_g[g], center_emb[g * BG:(g + 1) * BG],
            conf_t[:, g * BG:(g + 1) * BG], Wq, Wk, Wg1, Wg2,
            bg2, gamma2, beta2))
    return jnp.concatenate(outs, axis=0)
